# baseline (speedup 1.0000x reference)
import functools

import jax
import jax.numpy as jnp
from jax import lax
from jax.experimental import pallas as pl
from jax.experimental.pallas import tpu as pltpu
from jax.experimental.pallas import tpu_sc as plsc

_K = 2
_CAP = 2
_NC = 2
_NS = 16
_NW = _NC * _NS
_LANES = 16


def _sc_mesh():
    return plsc.VectorSubcoreMesh(
        core_axis_name="c", subcore_axis_name="s", num_cores=_NC,
        num_subcores=_NS)


def _wid():
    return lax.axis_index("s") * _NC + lax.axis_index("c")


def _combine(y2d, cidx0, cidx1, t, d):
    tok_w = t // _NW
    nvc = d // _LANES

    @functools.partial(
        pl.kernel,
        out_type=jax.ShapeDtypeStruct((t, d), jnp.float32),
        mesh=_sc_mesh(),
        scratch_types=[
            pltpu.VMEM((tok_w,), jnp.int32),
            pltpu.VMEM((tok_w,), jnp.int32),
            pltpu.VMEM((tok_w, d), jnp.float32),
            pltpu.VMEM((tok_w, d), jnp.float32),
            pltpu.SemaphoreType.DMA,
            pltpu.SemaphoreType.DMA,
        ],
    )
    def k(y_hbm, c0_hbm, c1_hbm, out_hbm, i0_v, i1_v, g0_v, g1_v, s0, s1):
        base = _wid() * tok_w
        pltpu.sync_copy(c0_hbm.at[pl.ds(base, tok_w)], i0_v)
        cp0 = pltpu.async_copy(y_hbm.at[i0_v], g0_v, s0)
        pltpu.sync_copy(c1_hbm.at[pl.ds(base, tok_w)], i1_v)
        cp1 = pltpu.async_copy(y_hbm.at[i1_v], g1_v, s1)
        cp0.wait()
        cp1.wait()

        def row_add(r, carry):
            for u in range(nvc):
                sl = pl.ds(u * _LANES, _LANES)
                g0_v[r, sl] = g0_v[r, sl] + g1_v[r, sl]
            return carry

        lax.fori_loop(0, tok_w, row_add, 0)
        pltpu.sync_copy(g0_v, out_hbm.at[pl.ds(base, tok_w)])

    return k(y2d, cidx0, cidx1)


def _ffn_body(n_e, t, c, e0_ref, e1_ref, x_ref, w1_ref, b1_ref, w2_ref,
              b2_ref, y_ref, c0_ref, c1_ref, p0_ref, p1_ref):
    e = pl.program_id(0)

    @pl.when(e == 0)
    def _positions():
        iota_e = lax.broadcasted_iota(jnp.int32, (t, n_e), 1)
        oh0 = (e0_ref[...] == iota_e).astype(jnp.int32)
        oh1 = (e1_ref[...] == iota_e).astype(jnp.int32)
        cnt = oh0 + oh1
        csum = cnt
        s = 1
        while s < t:
            shifted = jnp.concatenate(
                [jnp.zeros((s, n_e), jnp.int32), csum[: t - s, :]], axis=0)
            csum = csum + shifted
            s *= 2
        excl = csum - cnt
        p0 = jnp.sum(excl * oh0, axis=1, keepdims=True)
        p1 = (jnp.sum(excl * oh1, axis=1, keepdims=True)
              + (e0_ref[...] == e1_ref[...]).astype(jnp.int32))
        p0_ref[...] = p0
        p1_ref[...] = p1
        pad = n_e * c
        c0_ref[...] = jnp.where(p0 < c, e0_ref[...] * c + p0, pad)
        c1_ref[...] = jnp.where(p1 < c, e1_ref[...] * c + p1, pad)

    @pl.when(e < n_e)
    def _compute():
        iota_c = lax.broadcasted_iota(jnp.int32, (t, c), 1)
        oht = (((p0_ref[...] == iota_c) & (e0_ref[...] == e))
               .astype(jnp.bfloat16)
               + ((p1_ref[...] == iota_c) & (e1_ref[...] == e))
               .astype(jnp.bfloat16))
        xe = lax.dot_general(oht, x_ref[...], (((0,), (0,)), ((), ())),
                             preferred_element_type=jnp.float32)
        hm = jnp.dot(xe.astype(jnp.bfloat16), w1_ref[0].astype(jnp.bfloat16),
                     preferred_element_type=jnp.float32)
        hm = jax.nn.gelu(hm + b1_ref[0, 0, :])
        part = jnp.dot(hm.astype(jnp.bfloat16), w2_ref[0].astype(jnp.bfloat16),
                       preferred_element_type=jnp.float32)
        y_ref[0] = (part + b2_ref[0, 0, :]) * 0.5

    @pl.when(e == n_e)
    def _pad_zero():
        y_ref[0] = jnp.zeros_like(y_ref[0])


def _expert_ffn(e0c, e1c, x_bf, W1, b1r, W2, b2r, n_e, c, d, f, t):
    grid = (n_e + 1,)
    clamp = lambda e: jnp.minimum(e, n_e - 1)
    col = pl.BlockSpec((t, 1), lambda e: (0, 0))
    return pl.pallas_call(
        functools.partial(_ffn_body, n_e, t, c),
        grid=grid,
        in_specs=[
            col, col,
            pl.BlockSpec((t, d), lambda e: (0, 0)),
            pl.BlockSpec((1, d, f), lambda e: (clamp(e), 0, 0)),
            pl.BlockSpec((1, 1, f), lambda e: (clamp(e), 0, 0)),
            pl.BlockSpec((1, f, d), lambda e: (clamp(e), 0, 0)),
            pl.BlockSpec((1, 1, d), lambda e: (clamp(e), 0, 0)),
        ],
        out_specs=[
            pl.BlockSpec((1, c, d), lambda e: (e, 0, 0)),
            col, col,
        ],
        out_shape=[
            jax.ShapeDtypeStruct((n_e + 1, c, d), jnp.float32),
            jax.ShapeDtypeStruct((t, 1), jnp.int32),
            jax.ShapeDtypeStruct((t, 1), jnp.int32),
        ],
        scratch_shapes=[
            pltpu.VMEM((t, 1), jnp.int32),
            pltpu.VMEM((t, 1), jnp.int32),
        ],
    )(e0c, e1c, x_bf, W1, b1r, W2, b2r)


def kernel(hidden_states, hash_proj, W1, b1, W2, b2):
    B, S, D = hidden_states.shape
    E, _, F = W1.shape
    T = B * S
    C = _CAP * (-(-(T * _K) // E))

    x = hidden_states.reshape(T, D)

    h = x @ hash_proj.T
    buckets = jnp.mod(jnp.floor(jnp.abs(h) * 997.0).astype(jnp.int32), E)
    sel = buckets[:, :_K]
    e0c = sel[:, 0].reshape(T, 1)
    e1c = sel[:, 1].reshape(T, 1)

    x_bf = x.astype(jnp.bfloat16)
    b1r = b1.reshape(E, 1, F)
    b2r = b2.reshape(E, 1, D)
    y3, cidx0, cidx1 = _expert_ffn(
        e0c, e1c, x_bf, W1, b1r, W2, b2r, E, C, D, F, T)
    y2d = y3.reshape((E + 1) * C, D)
    out = _combine(y2d, cidx0.reshape(T), cidx1.reshape(T), T, D)
    return out.reshape(B, S, D)


# speedup vs baseline: 1.1145x; 1.1145x over previous
import functools

import jax
import jax.numpy as jnp
from jax import lax
from jax.experimental import pallas as pl
from jax.experimental.pallas import tpu as pltpu
from jax.experimental.pallas import tpu_sc as plsc

_K = 2
_CAP = 2
_NC = 2
_NS = 16
_NW = _NC * _NS
_LANES = 16
_BLK = 128


def _sc_mesh():
    return plsc.VectorSubcoreMesh(
        core_axis_name="c", subcore_axis_name="s", num_cores=_NC,
        num_subcores=_NS)


def _wid():
    return lax.axis_index("s") * _NC + lax.axis_index("c")


def _combine(y2d, cidx0, cidx1, t, d):
    tok_w = t // _NW
    nvc = d // _LANES

    @functools.partial(
        pl.kernel,
        out_type=jax.ShapeDtypeStruct((t, d), jnp.float32),
        mesh=_sc_mesh(),
        scratch_types=[
            pltpu.VMEM((tok_w,), jnp.int32),
            pltpu.VMEM((tok_w,), jnp.int32),
            pltpu.VMEM((tok_w, d), jnp.float32),
            pltpu.VMEM((tok_w, d), jnp.float32),
            pltpu.SemaphoreType.DMA,
            pltpu.SemaphoreType.DMA,
        ],
    )
    def k(y_hbm, c0_hbm, c1_hbm, out_hbm, i0_v, i1_v, g0_v, g1_v, s0, s1):
        base = _wid() * tok_w
        pltpu.sync_copy(c0_hbm.at[pl.ds(base, tok_w)], i0_v)
        cp0 = pltpu.async_copy(y_hbm.at[i0_v], g0_v, s0)
        pltpu.sync_copy(c1_hbm.at[pl.ds(base, tok_w)], i1_v)
        cp1 = pltpu.async_copy(y_hbm.at[i1_v], g1_v, s1)
        cp0.wait()
        cp1.wait()

        def row_add(r, carry):
            for u in range(nvc):
                sl = pl.ds(u * _LANES, _LANES)
                g0_v[r, sl] = g0_v[r, sl] + g1_v[r, sl]
            return carry

        lax.fori_loop(0, tok_w, row_add, 0)
        pltpu.sync_copy(g0_v, out_hbm.at[pl.ds(base, tok_w)])

    return k(y2d, cidx0, cidx1)


def _ffn_body(n_e, t, c, e0_ref, e1_ref, p0_ref, p1_ref, x_ref, w1_ref,
              b1_ref, w2_ref, b2_ref, y_ref):
    e = pl.program_id(0)

    @pl.when(e < n_e)
    def _compute():
        iota_c = lax.broadcasted_iota(jnp.int32, (c, t), 0)
        oht = (((p0_ref[...] == iota_c) & (e0_ref[...] == e))
               .astype(jnp.bfloat16)
               + ((p1_ref[...] == iota_c) & (e1_ref[...] == e))
               .astype(jnp.bfloat16))
        xe = jnp.dot(oht, x_ref[...], preferred_element_type=jnp.float32)
        hm = jnp.dot(xe.astype(jnp.bfloat16), w1_ref[0].astype(jnp.bfloat16),
                     preferred_element_type=jnp.float32)
        hm = jax.nn.gelu(hm + b1_ref[0, 0, :])
        part = jnp.dot(hm.astype(jnp.bfloat16), w2_ref[0].astype(jnp.bfloat16),
                       preferred_element_type=jnp.float32)
        y_ref[0] = (part + b2_ref[0, 0, :]) * 0.5

    @pl.when(e == n_e)
    def _pad_zero():
        y_ref[0] = jnp.zeros_like(y_ref[0])


def _expert_ffn(e0, e1, p0, p1, x_bf, W1, b1r, W2, b2r, n_e, c, d, f, t):
    grid = (n_e + 1,)
    clamp = lambda e: jnp.minimum(e, n_e - 1)
    row = pl.BlockSpec((1, t), lambda e: (0, 0))
    return pl.pallas_call(
        functools.partial(_ffn_body, n_e, t, c),
        grid=grid,
        in_specs=[
            row, row, row, row,
            pl.BlockSpec((t, d), lambda e: (0, 0)),
            pl.BlockSpec((1, d, f), lambda e: (clamp(e), 0, 0)),
            pl.BlockSpec((1, 1, f), lambda e: (clamp(e), 0, 0)),
            pl.BlockSpec((1, f, d), lambda e: (clamp(e), 0, 0)),
            pl.BlockSpec((1, 1, d), lambda e: (clamp(e), 0, 0)),
        ],
        out_specs=pl.BlockSpec((1, c, d), lambda e: (e, 0, 0)),
        out_shape=jax.ShapeDtypeStruct((n_e + 1, c, d), jnp.float32),
    )(e0, e1, p0, p1, x_bf, W1, b1r, W2, b2r)


def kernel(hidden_states, hash_proj, W1, b1, W2, b2):
    B, S, D = hidden_states.shape
    E, _, F = W1.shape
    T = B * S
    C = _CAP * (-(-(T * _K) // E))
    J = T * _K
    NB = J // _BLK

    x = hidden_states.reshape(T, D)

    h = x @ hash_proj.T
    buckets = jnp.mod(jnp.floor(jnp.abs(h) * 997.0).astype(jnp.int32), E)
    flat_e = buckets[:, :_K].reshape(-1)

    fe = flat_e.reshape(NB, _BLK)
    oh = (fe[:, :, None] == jnp.arange(E, dtype=jnp.int32)).astype(
        jnp.bfloat16)
    lt = jnp.tril(jnp.ones((_BLK, _BLK), jnp.bfloat16), k=-1)
    intra = jnp.einsum('rc,bce->bre', lt, oh,
                       preferred_element_type=jnp.float32)
    blktot = jnp.sum(oh.astype(jnp.float32), axis=1)
    off = jnp.cumsum(blktot, axis=0) - blktot
    excl = intra + off[:, None, :]
    pos = jnp.sum(excl * oh.astype(jnp.float32), axis=-1)
    pos = pos.reshape(J).astype(jnp.int32)

    ep = flat_e.reshape(T, _K)
    pp = pos.reshape(T, _K)
    e0 = ep[:, 0].reshape(1, T)
    e1 = ep[:, 1].reshape(1, T)
    p0 = pp[:, 0].reshape(1, T)
    p1 = pp[:, 1].reshape(1, T)

    cidx = jnp.where(pos < C, flat_e * C + pos, E * C).astype(jnp.int32)
    cidx2 = cidx.reshape(T, _K)
    cidx0 = cidx2[:, 0]
    cidx1 = cidx2[:, 1]

    x_bf = x.astype(jnp.bfloat16)
    b1r = b1.reshape(E, 1, F)
    b2r = b2.reshape(E, 1, D)
    y3 = _expert_ffn(e0, e1, p0, p1, x_bf, W1, b1r, W2, b2r, E, C, D, F, T)
    y2d = y3.reshape((E + 1) * C, D)
    out = _combine(y2d, cidx0, cidx1, T, D)
    return out.reshape(B, S, D)


# speedup vs baseline: 1.1269x; 1.0111x over previous
import functools

import jax
import jax.numpy as jnp
from jax import lax
from jax.experimental import pallas as pl
from jax.experimental.pallas import tpu as pltpu
from jax.experimental.pallas import tpu_sc as plsc

_K = 2
_CAP = 2
_NC = 2
_NS = 16
_NW = _NC * _NS
_LANES = 16
_BLK = 128


def _sc_mesh():
    return plsc.VectorSubcoreMesh(
        core_axis_name="c", subcore_axis_name="s", num_cores=_NC,
        num_subcores=_NS)


def _wid():
    return lax.axis_index("s") * _NC + lax.axis_index("c")


def _combine(y2d, cidx0, cidx1, t, d):
    tok_w = t // _NW
    nvc = d // _LANES

    @functools.partial(
        pl.kernel,
        out_type=jax.ShapeDtypeStruct((t, d), jnp.float32),
        mesh=_sc_mesh(),
        scratch_types=[
            pltpu.VMEM((tok_w,), jnp.int32),
            pltpu.VMEM((tok_w,), jnp.int32),
            pltpu.VMEM((tok_w, d), jnp.float32),
            pltpu.VMEM((tok_w, d), jnp.float32),
            pltpu.SemaphoreType.DMA,
            pltpu.SemaphoreType.DMA,
        ],
    )
    def k(y_hbm, c0_hbm, c1_hbm, out_hbm, i0_v, i1_v, g0_v, g1_v, s0, s1):
        base = _wid() * tok_w
        pltpu.sync_copy(c0_hbm.at[pl.ds(base, tok_w)], i0_v)
        cp0 = pltpu.async_copy(y_hbm.at[i0_v], g0_v, s0)
        pltpu.sync_copy(c1_hbm.at[pl.ds(base, tok_w)], i1_v)
        cp1 = pltpu.async_copy(y_hbm.at[i1_v], g1_v, s1)
        cp0.wait()
        cp1.wait()

        def row_add(r, carry):
            for u in range(nvc):
                sl = pl.ds(u * _LANES, _LANES)
                g0_v[r, sl] = g0_v[r, sl] + g1_v[r, sl]
            return carry

        lax.fori_loop(0, tok_w, row_add, 0)
        pltpu.sync_copy(g0_v, out_hbm.at[pl.ds(base, tok_w)])

    return k(y2d, cidx0, cidx1)


def _ffn_body(n_e, t, c, e0_ref, e1_ref, p0_ref, p1_ref, x_ref, w1_ref,
              b1_ref, w2_ref, b2_ref, y_ref, xbf_ref):
    e = pl.program_id(0)

    @pl.when(e == 0)
    def _cast_x():
        xbf_ref[...] = x_ref[...].astype(jnp.bfloat16)

    @pl.when(e < n_e)
    def _compute():
        iota_c = lax.broadcasted_iota(jnp.int32, (c, t), 0)
        oht = (((p0_ref[...] == iota_c) & (e0_ref[...] == e))
               .astype(jnp.bfloat16)
               + ((p1_ref[...] == iota_c) & (e1_ref[...] == e))
               .astype(jnp.bfloat16))
        xe = jnp.dot(oht, xbf_ref[...], preferred_element_type=jnp.float32)
        hm = jnp.dot(xe.astype(jnp.bfloat16), w1_ref[0].astype(jnp.bfloat16),
                     preferred_element_type=jnp.float32)
        hm = jax.nn.gelu(hm + b1_ref[0, 0, :])
        part = jnp.dot(hm.astype(jnp.bfloat16), w2_ref[0].astype(jnp.bfloat16),
                       preferred_element_type=jnp.float32)
        y_ref[0] = (part + b2_ref[0, 0, :]) * 0.5

    @pl.when(e == n_e)
    def _pad_zero():
        y_ref[0] = jnp.zeros_like(y_ref[0])


def _expert_ffn(e0, e1, p0, p1, x_bf, W1, b1r, W2, b2r, n_e, c, d, f, t):
    grid = (n_e + 1,)
    clamp = lambda e: jnp.minimum(e, n_e - 1)
    row = pl.BlockSpec((1, t), lambda e: (0, 0))
    return pl.pallas_call(
        functools.partial(_ffn_body, n_e, t, c),
        grid=grid,
        in_specs=[
            row, row, row, row,
            pl.BlockSpec((t, d), lambda e: (0, 0)),
            pl.BlockSpec((1, d, f), lambda e: (clamp(e), 0, 0)),
            pl.BlockSpec((1, 1, f), lambda e: (clamp(e), 0, 0)),
            pl.BlockSpec((1, f, d), lambda e: (clamp(e), 0, 0)),
            pl.BlockSpec((1, 1, d), lambda e: (clamp(e), 0, 0)),
        ],
        out_specs=pl.BlockSpec((1, c, d), lambda e: (e, 0, 0)),
        out_shape=jax.ShapeDtypeStruct((n_e + 1, c, d), jnp.float32),
        scratch_shapes=[pltpu.VMEM((t, d), jnp.bfloat16)],
    )(e0, e1, p0, p1, x_bf, W1, b1r, W2, b2r)


def kernel(hidden_states, hash_proj, W1, b1, W2, b2):
    B, S, D = hidden_states.shape
    E, _, F = W1.shape
    T = B * S
    C = _CAP * (-(-(T * _K) // E))
    J = T * _K
    NB = J // _BLK

    x = hidden_states.reshape(T, D)

    h = x @ hash_proj.T
    buckets = jnp.mod(jnp.floor(jnp.abs(h) * 997.0).astype(jnp.int32), E)
    flat_e = buckets[:, :_K].reshape(-1)

    fe = flat_e.reshape(NB, _BLK)
    oh = (fe[:, :, None] == jnp.arange(E, dtype=jnp.int32)).astype(
        jnp.bfloat16)
    lt = jnp.tril(jnp.ones((_BLK, _BLK), jnp.bfloat16), k=-1)
    intra = jnp.einsum('rc,bce->bre', lt, oh,
                       preferred_element_type=jnp.float32)
    blktot = jnp.sum(oh.astype(jnp.float32), axis=1)
    off = jnp.cumsum(blktot, axis=0) - blktot
    excl = intra + off[:, None, :]
    pos = jnp.sum(excl * oh.astype(jnp.float32), axis=-1)
    pos = pos.reshape(J).astype(jnp.int32)

    ep = flat_e.reshape(T, _K)
    pp = pos.reshape(T, _K)
    e0 = ep[:, 0].reshape(1, T)
    e1 = ep[:, 1].reshape(1, T)
    p0 = pp[:, 0].reshape(1, T)
    p1 = pp[:, 1].reshape(1, T)

    cidx = jnp.where(pos < C, flat_e * C + pos, E * C).astype(jnp.int32)
    cidx2 = cidx.reshape(T, _K)
    cidx0 = cidx2[:, 0]
    cidx1 = cidx2[:, 1]

    b1r = b1.reshape(E, 1, F)
    b2r = b2.reshape(E, 1, D)
    y3 = _expert_ffn(e0, e1, p0, p1, x, W1, b1r, W2, b2r, E, C, D, F, T)
    y2d = y3.reshape((E + 1) * C, D)
    out = _combine(y2d, cidx0, cidx1, T, D)
    return out.reshape(B, S, D)
